# baseline (speedup 1.0000x reference)
import functools

import jax
import jax.numpy as jnp
import numpy as np
from jax import lax
from jax.experimental import pallas as pl
from jax.experimental.pallas import tpu as pltpu
from jax.experimental.pallas import tpu_sc as plsc

NB, NA, ND = 100000, 200000, 300000
N_ATOM = 100000
NBP, NAP, NDP = 106496, 204800, 303104
NPOS = 100352
DUMMY = NPOS - 1
TOT = 2 * NBP + 3 * NAP + 4 * NDP
NW = 32
PER_W = TOT // NW
CHUNK = 3984
NCHUNK = PER_W // CHUNK
STRIPE = NPOS // 16

TOT2 = 2048000
SLICE = TOT2 // 10
CH2 = 6400
NCH2 = SLICE // CH2

@functools.cache
def _sc_kernels():
    mesh = plsc.VectorSubcoreMesh(core_axis_name="c", subcore_axis_name="s")
    cparams = pltpu.CompilerParams(use_tc_tiling_on_sc=False)
    cparams_nl = pltpu.CompilerParams(use_tc_tiling_on_sc=False,
                                      needs_layout_passes=False)
    f32 = jnp.float32

    @functools.partial(
        pl.kernel, mesh=mesh, compiler_params=cparams_nl,
        out_type=jax.ShapeDtypeStruct((3, TOT2), f32),
        scratch_types=(
            [pltpu.VMEM((NPOS,), f32)]
            + [pltpu.VMEM((CH2,), jnp.int32)] * 2
            + [pltpu.VMEM((CH2,), f32)] * 2
            + [pltpu.SemaphoreType.DMA] * 2
        ),
    )
    def sc_gather(tab_hbm, idx_hbm, out_hbm,
                  plane_v, idx0, idx1, ov0, ov1, sem_idx, sem_st):
        wid = lax.axis_index("s") * 2 + lax.axis_index("c")

        @pl.when(wid < 30)
        def _():
            comp = wid % 3
            sl_id = wid // 3
            base0 = sl_id * SLICE
            idxs = (idx0, idx1)
            ovs = (ov0, ov1)
            d_idx = [None, None]
            pend_st = [None, None]
            d_idx[0] = pltpu.async_copy(
                idx_hbm.at[pl.ds(base0, CH2)], idxs[0], sem_idx)
            pltpu.sync_copy(tab_hbm.at[comp], plane_v)
            for j in range(NCH2):
                b = j % 2
                d_idx[b].wait()
                if j + 1 < NCH2:
                    d_idx[1 - b] = pltpu.async_copy(
                        idx_hbm.at[pl.ds(base0 + (j + 1) * CH2, CH2)],
                        idxs[1 - b], sem_idx)
                if pend_st[b] is not None:
                    pend_st[b].wait()

                def body(i):
                    iv = idxs[b][pl.ds(i * 16, 16)]
                    ovs[b][pl.ds(i * 16, 16)] = plsc.load_gather(
                        plane_v, [iv])

                pl.loop(0, CH2 // 16, unroll=8)(body)
                pend_st[b] = pltpu.async_copy(
                    ovs[b], out_hbm.at[comp, pl.ds(base0 + j * CH2, CH2)],
                    sem_st)
            for b in range(2):
                if pend_st[b] is not None:
                    pend_st[b].wait()

    @functools.partial(
        pl.kernel, mesh=mesh, compiler_params=cparams,
        out_type=[jax.ShapeDtypeStruct((2 * NPOS,), f32)] * 3,
        scratch_types=(
            [pltpu.VMEM((CHUNK,), jnp.int32)] * 2
            + [pltpu.VMEM((CHUNK,), f32)] * 6
            + [pltpu.SemaphoreType.DMA] * 2
            + [pltpu.VMEM_SHARED((NPOS,), f32)] * 3
        ),
    )
    def sc_scatter(cx_hbm, cy_hbm, cz_hbm, idx_hbm, zeros_hbm,
                   ox_hbm, oy_hbm, oz_hbm,
                   idx0, idx1, xv0, xv1, yv0, yv1, zv0, zv1,
                   sem_ld, sem_add, gx_sh, gy_sh, gz_sh):
        cid = lax.axis_index("c")
        sid = lax.axis_index("s")
        wid = sid * 2 + cid
        base0 = wid * PER_W
        stripe = pl.ds(sid * STRIPE, STRIPE)
        idxs = (idx0, idx1)
        xvs, yvs, zvs = (xv0, xv1), (yv0, yv1), (zv0, zv1)

        def fire_loads(j, b):
            sl = pl.ds(base0 + j * CHUNK, CHUNK)
            return [
                pltpu.async_copy(idx_hbm.at[sl], idxs[b], sem_ld),
                pltpu.async_copy(cx_hbm.at[sl], xvs[b], sem_ld),
                pltpu.async_copy(cy_hbm.at[sl], yvs[b], sem_ld),
                pltpu.async_copy(cz_hbm.at[sl], zvs[b], sem_ld),
            ]

        pend_ld = [None, None]
        pend_add = [None, None]
        pend_ld[0] = fire_loads(0, 0)
        pltpu.sync_copy(zeros_hbm, gx_sh.at[stripe])
        pltpu.sync_copy(zeros_hbm, gy_sh.at[stripe])
        pltpu.sync_copy(zeros_hbm, gz_sh.at[stripe])
        plsc.subcore_barrier()
        for j in range(NCHUNK):
            b = j % 2
            for d in pend_ld[b]:
                d.wait()
            pend_add[b] = [
                pltpu.async_copy(xvs[b], gx_sh.at[idxs[b]], sem_add, add=True),
                pltpu.async_copy(yvs[b], gy_sh.at[idxs[b]], sem_add, add=True),
                pltpu.async_copy(zvs[b], gz_sh.at[idxs[b]], sem_add, add=True),
            ]
            if j + 1 < NCHUNK:
                if pend_add[1 - b] is not None:
                    for d in pend_add[1 - b]:
                        d.wait()
                pend_ld[1 - b] = fire_loads(j + 1, 1 - b)
        for b in range(2):
            if pend_add[b] is not None:
                for d in pend_add[b]:
                    d.wait()
        plsc.subcore_barrier()
        out_off = pl.ds(cid * NPOS + sid * STRIPE, STRIPE)
        pltpu.sync_copy(gx_sh.at[stripe], ox_hbm.at[out_off])
        pltpu.sync_copy(gy_sh.at[stripe], oy_hbm.at[out_off])
        pltpu.sync_copy(gz_sh.at[stripe], oz_hbm.at[out_off])

    return sc_gather, sc_scatter


def _sin_poly(t):
    t2 = t * t
    return t * (1.0 + t2 * (-1.0 / 6 + t2 * (1.0 / 120 + t2 * (-1.0 / 5040 + t2 / 362880))))


def _cos_poly(t):
    t2 = t * t
    return 1.0 + t2 * (-0.5 + t2 * (1.0 / 24 + t2 * (-1.0 / 720 + t2 * (1.0 / 40320 - t2 / 3628800))))


def _arccos_poly(c):
    t = jnp.abs(c)
    s = jnp.sqrt(1.0 - t)
    p = 1.5707288 + t * (-0.2121144 + t * (0.0742610 - 0.0187293 * t))
    r = s * p
    return jnp.where(c >= 0, r, np.pi - r)


def _bond_body(eq, tol, x0, y0, z0, x1, y1, z1, ox0, oy0, oz0, ox1, oy1, oz1):
    vx = x1[...] - x0[...]
    vy = y1[...] - y0[...]
    vz = z1[...] - z0[...]
    r = jnp.sqrt(vx * vx + vy * vy + vz * vz)
    ir = 1.0 / r
    dr = r - eq[...]
    t = tol[...]
    act = ((dr * dr - t * t) > 0).astype(jnp.float32)
    coef = (2000.0 / NB) * dr * act * ir
    ox1[...] = coef * vx
    oy1[...] = coef * vy
    oz1[...] = coef * vz
    ox0[...] = -coef * vx
    oy0[...] = -coef * vy
    oz0[...] = -coef * vz


def _angle_body(eq, tol, x0, y0, z0, x1, y1, z1, x2, y2, z2,
                o0x, o0y, o0z, o1x, o1y, o1z, o2x, o2y, o2z):
    b0x = x0[...] - x1[...]
    b0y = y0[...] - y1[...]
    b0z = z0[...] - z1[...]
    b1x = x2[...] - x1[...]
    b1y = y2[...] - y1[...]
    b1z = z2[...] - z1[...]
    n0 = jnp.sqrt(b0x * b0x + b0y * b0y + b0z * b0z)
    n1 = jnp.sqrt(b1x * b1x + b1y * b1y + b1z * b1z)
    d = b0x * b1x + b0y * b1y + b0z * b1z
    q = n0 * n1 + 1e-12
    c = d / q
    lo, hi = -1.0 + 1e-7, 1.0 - 1e-7
    c_cl = jnp.clip(c, lo, hi)
    theta = _arccos_poly(c_cl)
    dth = theta - eq[...]
    t = tol[...]
    act = ((dth * dth - t * t) > 0).astype(jnp.float32)
    gtheta = (300.0 / NA) * dth * act
    inside = ((c > lo) & (c < hi)).astype(jnp.float32)
    gc = -gtheta * lax.rsqrt(1.0 - c_cl * c_cl) * inside
    gd = gc / q
    gq = -gc * d / (q * q)
    f0 = gq * n1 / n0
    f1 = gq * n0 / n1
    g0x = gd * b1x + f0 * b0x
    g0y = gd * b1y + f0 * b0y
    g0z = gd * b1z + f0 * b0z
    g1x = gd * b0x + f1 * b1x
    g1y = gd * b0y + f1 * b1y
    g1z = gd * b0z + f1 * b1z
    o0x[...] = g0x
    o0y[...] = g0y
    o0z[...] = g0z
    o2x[...] = g1x
    o2y[...] = g1y
    o2z[...] = g1z
    o1x[...] = -(g0x + g1x)
    o1y[...] = -(g0y + g1y)
    o1z[...] = -(g0z + g1z)


def _dih_body(eq, x0, y0, z0, x1, y1, z1, x2, y2, z2, x3, y3, z3,
              o0x, o0y, o0z, o1x, o1y, o1z, o2x, o2y, o2z, o3x, o3y, o3z):
    b0x = x0[...] - x1[...]
    b0y = y0[...] - y1[...]
    b0z = z0[...] - z1[...]
    b1x = x2[...] - x1[...]
    b1y = y2[...] - y1[...]
    b1z = z2[...] - z1[...]
    b2x = x3[...] - x2[...]
    b2y = y3[...] - y2[...]
    b2z = z3[...] - z2[...]
    n1 = jnp.sqrt(b1x * b1x + b1y * b1y + b1z * b1z)
    inb = 1.0 / (n1 + 1e-12)
    ux, uy, uz = b1x * inb, b1y * inb, b1z * inb
    sv = b0x * ux + b0y * uy + b0z * uz
    vx_, vy_, vz_ = b0x - sv * ux, b0y - sv * uy, b0z - sv * uz
    sw = b2x * ux + b2y * uy + b2z * uz
    wx, wy, wz = b2x - sw * ux, b2y - sw * uy, b2z - sw * uz
    crx = uy * vz_ - uz * vy_
    cry = uz * vx_ - ux * vz_
    crz = ux * vy_ - uy * vx_
    x = vx_ * wx + vy_ * wy + vz_ * wz
    y = crx * wx + cry * wy + crz * wz
    den = x * x + y * y
    iden = 1.0 / den
    irho = lax.rsqrt(den)
    sphi = y * irho
    cphi = x * irho
    e = eq[...]
    seq = _sin_poly(e)
    ceq = _cos_poly(e)
    sdlt = sphi * ceq - cphi * seq
    gphi = (2.0 / ND) * sdlt
    gx = -y * iden * gphi
    gy = x * iden * gphi
    gvx, gvy, gvz = gx * wx, gx * wy, gx * wz
    gwx = gx * vx_ + gy * crx
    gwy = gx * vy_ + gy * cry
    gwz = gx * vz_ + gy * crz
    gcrx, gcry, gcrz = gy * wx, gy * wy, gy * wz
    gux = vy_ * gcrz - vz_ * gcry
    guy = vz_ * gcrx - vx_ * gcrz
    guz = vx_ * gcry - vy_ * gcrx
    gvx += gcry * uz - gcrz * uy
    gvy += gcrz * ux - gcrx * uz
    gvz += gcrx * uy - gcry * ux
    gb2x, gb2y, gb2z = gwx, gwy, gwz
    gsw = -(gwx * ux + gwy * uy + gwz * uz)
    gux -= sw * gwx
    guy -= sw * gwy
    guz -= sw * gwz
    gb2x += gsw * ux
    gb2y += gsw * uy
    gb2z += gsw * uz
    gux += gsw * b2x
    guy += gsw * b2y
    guz += gsw * b2z
    gb0x, gb0y, gb0z = gvx, gvy, gvz
    gsv = -(gvx * ux + gvy * uy + gvz * uz)
    gux -= sv * gvx
    guy -= sv * gvy
    guz -= sv * gvz
    gb0x += gsv * ux
    gb0y += gsv * uy
    gb0z += gsv * uz
    gux += gsv * b0x
    guy += gsv * b0y
    guz += gsv * b0z
    gb1x = gux * inb
    gb1y = guy * inb
    gb1z = guz * inb
    gn1 = -(gux * b1x + guy * b1y + guz * b1z) * inb * inb
    fin = gn1 / n1
    gb1x += fin * b1x
    gb1y += fin * b1y
    gb1z += fin * b1z
    o0x[...] = gb0x
    o0y[...] = gb0y
    o0z[...] = gb0z
    o1x[...] = -gb0x - gb1x
    o1y[...] = -gb0y - gb1y
    o1z[...] = -gb0z - gb1z
    o2x[...] = gb1x - gb2x
    o2y[...] = gb1y - gb2y
    o2z[...] = gb1z - gb2z
    o3x[...] = gb2x
    o3y[...] = gb2y
    o3z[...] = gb2z


def _update_body(dt_ref, px, py, pz, ax, ay, az, bx, by, bz, ox, oy, oz):
    dt = dt_ref[0]

    def clean(v):
        return jnp.where(jnp.isnan(v), 0.0, v)

    gx = clean(-(ax[...] + bx[...]))
    gy = clean(-(ay[...] + by[...]))
    gz = clean(-(az[...] + bz[...]))
    fn = jnp.sqrt(gx * gx + gy * gy + gz * gz)
    thresh = 0.1 / dt
    scale = jnp.where(fn > thresh, thresh / (fn + 1e-12), 1.0)
    row = lax.broadcasted_iota(jnp.int32, (98, 1024), 0)
    col = lax.broadcasted_iota(jnp.int32, (98, 1024), 1)
    mov = ((row * 1024 + col) < 50000).astype(jnp.float32) * dt
    ox[...] = px[...] + gx * scale * mov
    oy[...] = py[...] + gy * scale * mov
    oz[...] = pz[...] + gz * scale * mov


def _tc_call(body, grid_rows, n_in, n_out, *args):
    spec = pl.BlockSpec((8, 1024), lambda i: (i, 0))
    return pl.pallas_call(
        body,
        grid=(grid_rows // 8,),
        in_specs=[spec] * n_in,
        out_specs=[spec] * n_out,
        out_shape=[jax.ShapeDtypeStruct((grid_rows, 1024), jnp.float32)] * n_out,
    )(*args)


def _padcol(a, n, npad, val=0):
    return jnp.pad(a, (0, npad - n), constant_values=val)


def kernel(pos, bond_idcs, bond_eq_val, bond_tolerance, angle_idcs,
           angle_eq_val, angle_tolerance, dih_idcs, dih_eq_val,
           movable_pos_idcs, dtau):
    f32 = jnp.float32
    px = jnp.pad(pos[:, 0], (0, NPOS - N_ATOM))
    py = jnp.pad(pos[:, 1], (0, NPOS - N_ATOM))
    pz = jnp.pad(pos[:, 2], (0, NPOS - N_ATOM))

    idx_flat = jnp.concatenate([
        _padcol(bond_idcs[:, 0], NB, NBP, DUMMY),
        _padcol(bond_idcs[:, 1], NB, NBP, DUMMY),
        _padcol(angle_idcs[:, 0], NA, NAP, DUMMY),
        _padcol(angle_idcs[:, 1], NA, NAP, DUMMY),
        _padcol(angle_idcs[:, 2], NA, NAP, DUMMY),
        _padcol(dih_idcs[:, 0], ND, NDP, DUMMY),
        _padcol(dih_idcs[:, 1], ND, NDP, DUMMY),
        _padcol(dih_idcs[:, 2], ND, NDP, DUMMY),
        _padcol(dih_idcs[:, 3], ND, NDP, DUMMY),
    ]).astype(jnp.int32)

    sc_gather, sc_scatter = _sc_kernels()
    idx2 = jnp.pad(idx_flat, (0, TOT2 - TOT), constant_values=DUMMY)
    tab = jnp.stack([px, py, pz])
    gathered = sc_gather(tab, idx2)

    def plane(comp, off, cnt, rows):
        return lax.slice(
            gathered, (comp, off), (comp + 1, off + cnt)).reshape(rows, 1024)

    ob0, ob1 = 0, NBP
    oa0, oa1, oa2 = 2 * NBP, 2 * NBP + NAP, 2 * NBP + 2 * NAP
    od0 = 2 * NBP + 3 * NAP
    od1, od2, od3 = od0 + NDP, od0 + 2 * NDP, od0 + 3 * NDP

    beq = _padcol(bond_eq_val, NB, NBP).reshape(104, 1024)
    btol = _padcol(bond_tolerance, NB, NBP).reshape(104, 1024)
    bond_in = [beq, btol]
    for off in (ob0, ob1):
        for c in range(3):
            bond_in.append(plane(c, off, NBP, 104))
    bond_out = _tc_call(_bond_body, 104, 8, 6, *bond_in)

    aeq = _padcol(angle_eq_val, NA, NAP).reshape(200, 1024)
    atol = _padcol(angle_tolerance, NA, NAP).reshape(200, 1024)
    angle_in = [aeq, atol]
    for off in (oa0, oa1, oa2):
        for c in range(3):
            angle_in.append(plane(c, off, NAP, 200))
    angle_out = _tc_call(_angle_body, 200, 11, 9, *angle_in)

    deq = _padcol(dih_eq_val, ND, NDP).reshape(296, 1024)
    dih_in = [deq]
    for off in (od0, od1, od2, od3):
        for c in range(3):
            dih_in.append(plane(c, off, NDP, 296))
    dih_out = _tc_call(_dih_body, 296, 13, 12, *dih_in)

    planes = []
    for c in range(3):
        planes.append(jnp.concatenate([
            bond_out[0 + c].ravel(), bond_out[3 + c].ravel(),
            angle_out[0 + c].ravel(), angle_out[3 + c].ravel(),
            angle_out[6 + c].ravel(),
            dih_out[0 + c].ravel(), dih_out[3 + c].ravel(),
            dih_out[6 + c].ravel(), dih_out[9 + c].ravel(),
        ]))

    zeros_stripe = jnp.zeros((STRIPE,), f32)
    gx2, gy2, gz2 = sc_scatter(planes[0], planes[1], planes[2],
                               idx_flat, zeros_stripe)

    outs = pl.pallas_call(
        _update_body,
        in_specs=[pl.BlockSpec(memory_space=pltpu.SMEM)] + [pl.BlockSpec()] * 9,
        out_shape=[jax.ShapeDtypeStruct((98, 1024), f32)] * 3,
    )(dtau,
      px.reshape(98, 1024), py.reshape(98, 1024), pz.reshape(98, 1024),
      gx2[:NPOS].reshape(98, 1024), gy2[:NPOS].reshape(98, 1024),
      gz2[:NPOS].reshape(98, 1024),
      gx2[NPOS:].reshape(98, 1024), gy2[NPOS:].reshape(98, 1024),
      gz2[NPOS:].reshape(98, 1024))
    new_pos = jnp.stack(outs).reshape(3, NPOS).T[:N_ATOM]
    return new_pos


# speedup vs baseline: 2.3276x; 2.3276x over previous
import functools

import jax
import jax.numpy as jnp
import numpy as np
from jax import lax
from jax.experimental import pallas as pl
from jax.experimental.pallas import tpu as pltpu
from jax.experimental.pallas import tpu_sc as plsc

NB, NA, ND = 100000, 200000, 300000
N_ATOM = 100000
NBP, NAP, NDP = 106496, 204800, 303104
NPOS = 100352
DUMMY = NPOS - 1
TOT = 2 * NBP + 3 * NAP + 4 * NDP
NW = 32
PER_W = TOT // NW
CHUNK = 3984
NCHUNK = PER_W // CHUNK
STRIPE = NPOS // 16

TOT2 = 2048000
SLICE = TOT2 // 10
CH2 = 6400
NCH2 = SLICE // CH2

@functools.cache
def _sc_kernels():
    mesh = plsc.VectorSubcoreMesh(core_axis_name="c", subcore_axis_name="s")
    cparams = pltpu.CompilerParams(use_tc_tiling_on_sc=False)
    cparams_nl = pltpu.CompilerParams(use_tc_tiling_on_sc=False,
                                      needs_layout_passes=False)
    f32 = jnp.float32

    @functools.partial(
        pl.kernel, mesh=mesh, compiler_params=cparams_nl,
        out_type=[jax.ShapeDtypeStruct((TOT2,), f32)] * 3,
        scratch_types=(
            [pltpu.VMEM((NPOS,), f32)]
            + [pltpu.VMEM((CH2,), jnp.int32)] * 2
            + [pltpu.VMEM((CH2,), f32)] * 2
            + [pltpu.SemaphoreType.DMA] * 2
        ),
    )
    def sc_gather(tab_hbm, idx_hbm, ox_hbm, oy_hbm, oz_hbm,
                  plane_v, idx0, idx1, ov0, ov1, sem_idx, sem_st):
        wid = lax.axis_index("s") * 2 + lax.axis_index("c")

        @pl.when(wid < 30)
        def _():
            comp = wid % 3
            sl_id = wid // 3
            base0 = sl_id * SLICE
            idxs = (idx0, idx1)
            ovs = (ov0, ov1)
            d_idx = [None, None]
            pend_st = [None, None]
            d_idx[0] = pltpu.async_copy(
                idx_hbm.at[pl.ds(base0, CH2)], idxs[0], sem_idx)
            pltpu.sync_copy(tab_hbm.at[comp], plane_v)
            for j in range(NCH2):
                b = j % 2
                d_idx[b].wait()
                if j + 1 < NCH2:
                    d_idx[1 - b] = pltpu.async_copy(
                        idx_hbm.at[pl.ds(base0 + (j + 1) * CH2, CH2)],
                        idxs[1 - b], sem_idx)
                if pend_st[b] is not None:
                    pend_st[b][0].wait()

                def body(i):
                    iv = idxs[b][pl.ds(i * 16, 16)]
                    ovs[b][pl.ds(i * 16, 16)] = plsc.load_gather(
                        plane_v, [iv])

                pl.loop(0, CH2 // 16, unroll=8)(body)
                osl = pl.ds(base0 + j * CH2, CH2)
                d0 = pltpu.make_async_copy(ovs[b], ox_hbm.at[osl], sem_st)
                d1 = pltpu.make_async_copy(ovs[b], oy_hbm.at[osl], sem_st)
                d2 = pltpu.make_async_copy(ovs[b], oz_hbm.at[osl], sem_st)
                descs = (d0, d1, d2)

                @pl.when(comp == 0)
                def _():
                    d0.start()

                @pl.when(comp == 1)
                def _():
                    d1.start()

                @pl.when(comp == 2)
                def _():
                    d2.start()

                pend_st[b] = descs
            for b in range(2):
                if pend_st[b] is not None:
                    pend_st[b][0].wait()

    @functools.partial(
        pl.kernel, mesh=mesh, compiler_params=cparams,
        out_type=[jax.ShapeDtypeStruct((2 * NPOS,), f32)] * 3,
        scratch_types=(
            [pltpu.VMEM((CHUNK,), jnp.int32)] * 2
            + [pltpu.VMEM((CHUNK,), f32)] * 6
            + [pltpu.SemaphoreType.DMA] * 2
            + [pltpu.VMEM_SHARED((NPOS,), f32)] * 3
        ),
    )
    def sc_scatter(cx_hbm, cy_hbm, cz_hbm, idx_hbm, zeros_hbm,
                   ox_hbm, oy_hbm, oz_hbm,
                   idx0, idx1, xv0, xv1, yv0, yv1, zv0, zv1,
                   sem_ld, sem_add, gx_sh, gy_sh, gz_sh):
        cid = lax.axis_index("c")
        sid = lax.axis_index("s")
        wid = sid * 2 + cid
        base0 = wid * PER_W
        stripe = pl.ds(sid * STRIPE, STRIPE)
        idxs = (idx0, idx1)
        xvs, yvs, zvs = (xv0, xv1), (yv0, yv1), (zv0, zv1)

        def fire_loads(j, b):
            sl = pl.ds(base0 + j * CHUNK, CHUNK)
            return [
                pltpu.async_copy(idx_hbm.at[sl], idxs[b], sem_ld),
                pltpu.async_copy(cx_hbm.at[sl], xvs[b], sem_ld),
                pltpu.async_copy(cy_hbm.at[sl], yvs[b], sem_ld),
                pltpu.async_copy(cz_hbm.at[sl], zvs[b], sem_ld),
            ]

        pend_ld = [None, None]
        pend_add = [None, None]
        pend_ld[0] = fire_loads(0, 0)
        pltpu.sync_copy(zeros_hbm, gx_sh.at[stripe])
        pltpu.sync_copy(zeros_hbm, gy_sh.at[stripe])
        pltpu.sync_copy(zeros_hbm, gz_sh.at[stripe])
        plsc.subcore_barrier()
        for j in range(NCHUNK):
            b = j % 2
            for d in pend_ld[b]:
                d.wait()
            pend_add[b] = [
                pltpu.async_copy(xvs[b], gx_sh.at[idxs[b]], sem_add, add=True),
                pltpu.async_copy(yvs[b], gy_sh.at[idxs[b]], sem_add, add=True),
                pltpu.async_copy(zvs[b], gz_sh.at[idxs[b]], sem_add, add=True),
            ]
            if j + 1 < NCHUNK:
                if pend_add[1 - b] is not None:
                    for d in pend_add[1 - b]:
                        d.wait()
                pend_ld[1 - b] = fire_loads(j + 1, 1 - b)
        for b in range(2):
            if pend_add[b] is not None:
                for d in pend_add[b]:
                    d.wait()
        plsc.subcore_barrier()
        out_off = pl.ds(cid * NPOS + sid * STRIPE, STRIPE)
        pltpu.sync_copy(gx_sh.at[stripe], ox_hbm.at[out_off])
        pltpu.sync_copy(gy_sh.at[stripe], oy_hbm.at[out_off])
        pltpu.sync_copy(gz_sh.at[stripe], oz_hbm.at[out_off])

    return sc_gather, sc_scatter


def _sin_poly(t):
    t2 = t * t
    return t * (1.0 + t2 * (-1.0 / 6 + t2 * (1.0 / 120 + t2 * (-1.0 / 5040 + t2 / 362880))))


def _cos_poly(t):
    t2 = t * t
    return 1.0 + t2 * (-0.5 + t2 * (1.0 / 24 + t2 * (-1.0 / 720 + t2 * (1.0 / 40320 - t2 / 3628800))))


def _arccos_poly(c):
    t = jnp.abs(c)
    s = jnp.sqrt(1.0 - t)
    p = 1.5707288 + t * (-0.2121144 + t * (0.0742610 - 0.0187293 * t))
    r = s * p
    return jnp.where(c >= 0, r, np.pi - r)


def _bond_body(eq, tol, x0, y0, z0, x1, y1, z1, ox0, oy0, oz0, ox1, oy1, oz1):
    vx = x1[...] - x0[...]
    vy = y1[...] - y0[...]
    vz = z1[...] - z0[...]
    r = jnp.sqrt(vx * vx + vy * vy + vz * vz)
    ir = 1.0 / r
    dr = r - eq[...]
    t = tol[...]
    act = ((dr * dr - t * t) > 0).astype(jnp.float32)
    coef = (2000.0 / NB) * dr * act * ir
    ox1[...] = coef * vx
    oy1[...] = coef * vy
    oz1[...] = coef * vz
    ox0[...] = -coef * vx
    oy0[...] = -coef * vy
    oz0[...] = -coef * vz


def _angle_body(eq, tol, x0, y0, z0, x1, y1, z1, x2, y2, z2,
                o0x, o0y, o0z, o1x, o1y, o1z, o2x, o2y, o2z):
    b0x = x0[...] - x1[...]
    b0y = y0[...] - y1[...]
    b0z = z0[...] - z1[...]
    b1x = x2[...] - x1[...]
    b1y = y2[...] - y1[...]
    b1z = z2[...] - z1[...]
    n0 = jnp.sqrt(b0x * b0x + b0y * b0y + b0z * b0z)
    n1 = jnp.sqrt(b1x * b1x + b1y * b1y + b1z * b1z)
    d = b0x * b1x + b0y * b1y + b0z * b1z
    q = n0 * n1 + 1e-12
    c = d / q
    lo, hi = -1.0 + 1e-7, 1.0 - 1e-7
    c_cl = jnp.clip(c, lo, hi)
    theta = _arccos_poly(c_cl)
    dth = theta - eq[...]
    t = tol[...]
    act = ((dth * dth - t * t) > 0).astype(jnp.float32)
    gtheta = (300.0 / NA) * dth * act
    inside = ((c > lo) & (c < hi)).astype(jnp.float32)
    gc = -gtheta * lax.rsqrt(1.0 - c_cl * c_cl) * inside
    gd = gc / q
    gq = -gc * d / (q * q)
    f0 = gq * n1 / n0
    f1 = gq * n0 / n1
    g0x = gd * b1x + f0 * b0x
    g0y = gd * b1y + f0 * b0y
    g0z = gd * b1z + f0 * b0z
    g1x = gd * b0x + f1 * b1x
    g1y = gd * b0y + f1 * b1y
    g1z = gd * b0z + f1 * b1z
    o0x[...] = g0x
    o0y[...] = g0y
    o0z[...] = g0z
    o2x[...] = g1x
    o2y[...] = g1y
    o2z[...] = g1z
    o1x[...] = -(g0x + g1x)
    o1y[...] = -(g0y + g1y)
    o1z[...] = -(g0z + g1z)


def _dih_body(eq, x0, y0, z0, x1, y1, z1, x2, y2, z2, x3, y3, z3,
              o0x, o0y, o0z, o1x, o1y, o1z, o2x, o2y, o2z, o3x, o3y, o3z):
    b0x = x0[...] - x1[...]
    b0y = y0[...] - y1[...]
    b0z = z0[...] - z1[...]
    b1x = x2[...] - x1[...]
    b1y = y2[...] - y1[...]
    b1z = z2[...] - z1[...]
    b2x = x3[...] - x2[...]
    b2y = y3[...] - y2[...]
    b2z = z3[...] - z2[...]
    n1 = jnp.sqrt(b1x * b1x + b1y * b1y + b1z * b1z)
    inb = 1.0 / (n1 + 1e-12)
    ux, uy, uz = b1x * inb, b1y * inb, b1z * inb
    sv = b0x * ux + b0y * uy + b0z * uz
    vx_, vy_, vz_ = b0x - sv * ux, b0y - sv * uy, b0z - sv * uz
    sw = b2x * ux + b2y * uy + b2z * uz
    wx, wy, wz = b2x - sw * ux, b2y - sw * uy, b2z - sw * uz
    crx = uy * vz_ - uz * vy_
    cry = uz * vx_ - ux * vz_
    crz = ux * vy_ - uy * vx_
    x = vx_ * wx + vy_ * wy + vz_ * wz
    y = crx * wx + cry * wy + crz * wz
    den = x * x + y * y
    iden = 1.0 / den
    irho = lax.rsqrt(den)
    sphi = y * irho
    cphi = x * irho
    e = eq[...]
    seq = _sin_poly(e)
    ceq = _cos_poly(e)
    sdlt = sphi * ceq - cphi * seq
    gphi = (2.0 / ND) * sdlt
    gx = -y * iden * gphi
    gy = x * iden * gphi
    gvx, gvy, gvz = gx * wx, gx * wy, gx * wz
    gwx = gx * vx_ + gy * crx
    gwy = gx * vy_ + gy * cry
    gwz = gx * vz_ + gy * crz
    gcrx, gcry, gcrz = gy * wx, gy * wy, gy * wz
    gux = vy_ * gcrz - vz_ * gcry
    guy = vz_ * gcrx - vx_ * gcrz
    guz = vx_ * gcry - vy_ * gcrx
    gvx += gcry * uz - gcrz * uy
    gvy += gcrz * ux - gcrx * uz
    gvz += gcrx * uy - gcry * ux
    gb2x, gb2y, gb2z = gwx, gwy, gwz
    gsw = -(gwx * ux + gwy * uy + gwz * uz)
    gux -= sw * gwx
    guy -= sw * gwy
    guz -= sw * gwz
    gb2x += gsw * ux
    gb2y += gsw * uy
    gb2z += gsw * uz
    gux += gsw * b2x
    guy += gsw * b2y
    guz += gsw * b2z
    gb0x, gb0y, gb0z = gvx, gvy, gvz
    gsv = -(gvx * ux + gvy * uy + gvz * uz)
    gux -= sv * gvx
    guy -= sv * gvy
    guz -= sv * gvz
    gb0x += gsv * ux
    gb0y += gsv * uy
    gb0z += gsv * uz
    gux += gsv * b0x
    guy += gsv * b0y
    guz += gsv * b0z
    gb1x = gux * inb
    gb1y = guy * inb
    gb1z = guz * inb
    gn1 = -(gux * b1x + guy * b1y + guz * b1z) * inb * inb
    fin = gn1 / n1
    gb1x += fin * b1x
    gb1y += fin * b1y
    gb1z += fin * b1z
    o0x[...] = gb0x
    o0y[...] = gb0y
    o0z[...] = gb0z
    o1x[...] = -gb0x - gb1x
    o1y[...] = -gb0y - gb1y
    o1z[...] = -gb0z - gb1z
    o2x[...] = gb1x - gb2x
    o2y[...] = gb1y - gb2y
    o2z[...] = gb1z - gb2z
    o3x[...] = gb2x
    o3y[...] = gb2y
    o3z[...] = gb2z


def _update_body(dt_ref, px, py, pz, ax, ay, az, bx, by, bz, ox, oy, oz):
    dt = dt_ref[0]

    def clean(v):
        return jnp.where(jnp.isnan(v), 0.0, v)

    gx = clean(-(ax[...] + bx[...]))
    gy = clean(-(ay[...] + by[...]))
    gz = clean(-(az[...] + bz[...]))
    fn = jnp.sqrt(gx * gx + gy * gy + gz * gz)
    thresh = 0.1 / dt
    scale = jnp.where(fn > thresh, thresh / (fn + 1e-12), 1.0)
    row = lax.broadcasted_iota(jnp.int32, (98, 1024), 0)
    col = lax.broadcasted_iota(jnp.int32, (98, 1024), 1)
    mov = ((row * 1024 + col) < 50000).astype(jnp.float32) * dt
    ox[...] = px[...] + gx * scale * mov
    oy[...] = py[...] + gy * scale * mov
    oz[...] = pz[...] + gz * scale * mov


_PSPEC = pl.BlockSpec((8, 1024), lambda i: (i, 0))


def _gspec(off):
    r0 = off // 8192
    return pl.BlockSpec((8, 1024), lambda i, r=r0: (r + i, 0))


def _tc_call(body, grid_rows, in_specs, n_out, *args):
    return pl.pallas_call(
        body,
        grid=(grid_rows // 8,),
        in_specs=in_specs,
        out_specs=[_PSPEC] * n_out,
        out_shape=[jax.ShapeDtypeStruct((grid_rows, 1024), jnp.float32)] * n_out,
    )(*args)


def _padcol(a, n, npad, val=0):
    return jnp.pad(a, (0, npad - n), constant_values=val)


def kernel(pos, bond_idcs, bond_eq_val, bond_tolerance, angle_idcs,
           angle_eq_val, angle_tolerance, dih_idcs, dih_eq_val,
           movable_pos_idcs, dtau):
    f32 = jnp.float32
    px = jnp.pad(pos[:, 0], (0, NPOS - N_ATOM))
    py = jnp.pad(pos[:, 1], (0, NPOS - N_ATOM))
    pz = jnp.pad(pos[:, 2], (0, NPOS - N_ATOM))

    idx_flat = jnp.concatenate([
        _padcol(bond_idcs[:, 0], NB, NBP, DUMMY),
        _padcol(bond_idcs[:, 1], NB, NBP, DUMMY),
        _padcol(angle_idcs[:, 0], NA, NAP, DUMMY),
        _padcol(angle_idcs[:, 1], NA, NAP, DUMMY),
        _padcol(angle_idcs[:, 2], NA, NAP, DUMMY),
        _padcol(dih_idcs[:, 0], ND, NDP, DUMMY),
        _padcol(dih_idcs[:, 1], ND, NDP, DUMMY),
        _padcol(dih_idcs[:, 2], ND, NDP, DUMMY),
        _padcol(dih_idcs[:, 3], ND, NDP, DUMMY),
    ]).astype(jnp.int32)

    sc_gather, sc_scatter = _sc_kernels()
    idx2 = jnp.pad(idx_flat, (0, TOT2 - TOT), constant_values=DUMMY)
    tab = jnp.stack([px, py, pz])
    xs, ys, zs = sc_gather(tab, idx2)
    comps2d = tuple(a.reshape(TOT2 // 1024, 1024) for a in (xs, ys, zs))

    ob0, ob1 = 0, NBP
    oa0, oa1, oa2 = 2 * NBP, 2 * NBP + NAP, 2 * NBP + 2 * NAP
    od0 = 2 * NBP + 3 * NAP
    od1, od2, od3 = od0 + NDP, od0 + 2 * NDP, od0 + 3 * NDP

    beq = _padcol(bond_eq_val, NB, NBP).reshape(104, 1024)
    btol = _padcol(bond_tolerance, NB, NBP).reshape(104, 1024)
    bond_in, bond_specs = [beq, btol], [_PSPEC, _PSPEC]
    for off in (ob0, ob1):
        for c in range(3):
            bond_in.append(comps2d[c])
            bond_specs.append(_gspec(off))
    bond_out = _tc_call(_bond_body, 104, bond_specs, 6, *bond_in)

    aeq = _padcol(angle_eq_val, NA, NAP).reshape(200, 1024)
    atol = _padcol(angle_tolerance, NA, NAP).reshape(200, 1024)
    angle_in, angle_specs = [aeq, atol], [_PSPEC, _PSPEC]
    for off in (oa0, oa1, oa2):
        for c in range(3):
            angle_in.append(comps2d[c])
            angle_specs.append(_gspec(off))
    angle_out = _tc_call(_angle_body, 200, angle_specs, 9, *angle_in)

    deq = _padcol(dih_eq_val, ND, NDP).reshape(296, 1024)
    dih_in, dih_specs = [deq], [_PSPEC]
    for off in (od0, od1, od2, od3):
        for c in range(3):
            dih_in.append(comps2d[c])
            dih_specs.append(_gspec(off))
    dih_out = _tc_call(_dih_body, 296, dih_specs, 12, *dih_in)

    planes = []
    for c in range(3):
        planes.append(jnp.concatenate([
            bond_out[0 + c].ravel(), bond_out[3 + c].ravel(),
            angle_out[0 + c].ravel(), angle_out[3 + c].ravel(),
            angle_out[6 + c].ravel(),
            dih_out[0 + c].ravel(), dih_out[3 + c].ravel(),
            dih_out[6 + c].ravel(), dih_out[9 + c].ravel(),
        ]))

    zeros_stripe = jnp.zeros((STRIPE,), f32)
    gx2, gy2, gz2 = sc_scatter(planes[0], planes[1], planes[2],
                               idx_flat, zeros_stripe)

    outs = pl.pallas_call(
        _update_body,
        in_specs=[pl.BlockSpec(memory_space=pltpu.SMEM)] + [pl.BlockSpec()] * 9,
        out_shape=[jax.ShapeDtypeStruct((98, 1024), f32)] * 3,
    )(dtau,
      px.reshape(98, 1024), py.reshape(98, 1024), pz.reshape(98, 1024),
      gx2[:NPOS].reshape(98, 1024), gy2[:NPOS].reshape(98, 1024),
      gz2[:NPOS].reshape(98, 1024),
      gx2[NPOS:].reshape(98, 1024), gy2[NPOS:].reshape(98, 1024),
      gz2[NPOS:].reshape(98, 1024))
    new_pos = jnp.stack(outs).reshape(3, NPOS).T[:N_ATOM]
    return new_pos
